# baseline (speedup 1.0000x reference)
import functools

import jax
import jax.numpy as jnp
from jax import lax
from jax.experimental import pallas as pl
from jax.experimental.pallas import tpu as pltpu
from jax.experimental.pallas import tpu_sc as plsc

NUM_EMB = 1024
DIM = 64
B, S = 16, 576
ROWS = B * S
CHUNKS = ((12, 0), (4, 3))
GDIM = 128


def _pow2_pad(n):
    p = 128
    while p < n:
        p *= 2
    return p


def _prep_body(embT_ref, aux_ref, emb2T_ref, pad_ref):
    eT = embT_ref[...]
    se_row = jnp.sum(eT * eT, axis=0, keepdims=True)
    aux_ref[:, 0:1] = se_row.reshape(NUM_EMB, 1)
    aux_ref[:, 1:2] = lax.broadcasted_iota(
        jnp.int32, (NUM_EMB, 1), 0).astype(jnp.float32)
    emb2T_ref[...] = eT * (-2.0)
    pad_ref[:, 0:DIM] = jnp.swapaxes(eT, 0, 1)


def _prep(embT):
    return pl.pallas_call(
        _prep_body,
        out_shape=[jax.ShapeDtypeStruct((NUM_EMB, 128), jnp.float32),
                   jax.ShapeDtypeStruct((DIM, NUM_EMB), jnp.float32),
                   jax.ShapeDtypeStruct((NUM_EMB, GDIM), jnp.float32)],
    )(embT)


def _make_argmin_body(bch, idxpad):
    def _argmin_body(xt_ref, aux_ref, emb2T_ref, idx_ref):
        se = aux_ref[:, 0:1]
        iota = aux_ref[:, 1:2]
        for b in range(bch):
            xb = xt_ref[b]
            sx = jnp.sum(xb * xb, axis=0, keepdims=True)
            dot2 = lax.dot_general(
                emb2T_ref[...], xb, (((0,), (0,)), ((), ())),
                preferred_element_type=jnp.float32)
            d = (sx + se) + dot2
            m = jnp.min(d, axis=0, keepdims=True)
            cand = jnp.where(d == m, iota, float(NUM_EMB))
            idx_ref[pl.ds(b * S, S)] = (
                jnp.min(cand, axis=0).astype(jnp.int32))
    return _argmin_body


def _chunk_indices(xt, aux, emb2T, bch, blkidx):
    idxpad = _pow2_pad(bch * S)
    return pl.pallas_call(
        _make_argmin_body(bch, idxpad),
        grid=(1,),
        in_specs=[
            pl.BlockSpec((bch, DIM, S), lambda i, c=blkidx: (c, 0, 0)),
            pl.BlockSpec((NUM_EMB, 128), lambda i: (0, 0)),
            pl.BlockSpec((DIM, NUM_EMB), lambda i: (0, 0)),
        ],
        out_specs=pl.BlockSpec((idxpad,), lambda i: (0,)),
        out_shape=jax.ShapeDtypeStruct((idxpad,), jnp.int32),
    )(xt, aux, emb2T)


@functools.cache
def _make_gather(rows):
    info = plsc.get_sparse_core_info()
    nw = info.num_cores * info.num_subcores
    b_per_w = rows // nw

    @functools.partial(
        pl.kernel,
        out_type=jax.ShapeDtypeStruct((rows, GDIM), jnp.float32),
        mesh=plsc.VectorSubcoreMesh(core_axis_name="c", subcore_axis_name="s"),
        scratch_types=[
            pltpu.VMEM((b_per_w,), jnp.int32),
            pltpu.VMEM((b_per_w, GDIM), jnp.float32),
            pltpu.SemaphoreType.DMA,
        ],
    )
    def _gather_rows(emb_hbm, idx_hbm, out_hbm, idx_v, rows_v, sem):
        wid = lax.axis_index("s") * info.num_cores + lax.axis_index("c")
        base = wid * b_per_w
        pltpu.sync_copy(idx_hbm.at[pl.ds(base, b_per_w)], idx_v)
        pltpu.async_copy(emb_hbm.at[idx_v], rows_v, sem).wait()
        pltpu.sync_copy(rows_v, out_hbm.at[pl.ds(base, b_per_w)])

    return _gather_rows


def kernel(inputs, emb_weight):
    xt = jnp.swapaxes(inputs, 1, 2)
    embT = jnp.swapaxes(emb_weight, 0, 1)
    aux, emb2T, emb_pad = _prep(embT)
    idxs, quants = [], []
    for bch, blkidx in CHUNKS:
        rows = bch * S
        idx_c = _chunk_indices(xt, aux, emb2T, bch, blkidx)
        idxs.append(idx_c[:rows])
        quants.append(_make_gather(rows)(emb_pad, idx_c))
    idx = jnp.concatenate(idxs).reshape(B, S)
    q64 = jnp.concatenate(quants)[:, :DIM]
    quantized = q64.reshape(B, S, DIM)
    return (quantized, idx)


# speedup vs baseline: 1.0947x; 1.0947x over previous
import functools

import jax
import jax.numpy as jnp
from jax import lax
from jax.experimental import pallas as pl
from jax.experimental.pallas import tpu as pltpu
from jax.experimental.pallas import tpu_sc as plsc

NUM_EMB = 1024
DIM = 64
B, S = 16, 576
ROWS = B * S
NCHUNK = 2
BCH = B // NCHUNK
BLK = BCH * S
IDXPAD = 8192
GDIM = 128


def _prep_body(embT_ref, aux_ref, emb2T_ref, pad_ref):
    eT = embT_ref[...]
    se_row = jnp.sum(eT * eT, axis=0, keepdims=True)
    aux_ref[:, 0:1] = se_row.reshape(NUM_EMB, 1)
    aux_ref[:, 1:2] = lax.broadcasted_iota(
        jnp.int32, (NUM_EMB, 1), 0).astype(jnp.float32)
    emb2T_ref[...] = eT * (-2.0)
    pad_ref[:, 0:DIM] = jnp.swapaxes(eT, 0, 1)


def _prep(embT):
    return pl.pallas_call(
        _prep_body,
        out_shape=[jax.ShapeDtypeStruct((NUM_EMB, 128), jnp.float32),
                   jax.ShapeDtypeStruct((DIM, NUM_EMB), jnp.float32),
                   jax.ShapeDtypeStruct((NUM_EMB, GDIM), jnp.float32)],
    )(embT)


def _argmin_body(xt_ref, aux_ref, emb2T_ref, idx_ref):
    se = aux_ref[:, 0:1]
    iota = aux_ref[:, 1:2]
    xall = jnp.concatenate([xt_ref[b] for b in range(BCH)], axis=1)
    sx = jnp.sum(xall * xall, axis=0, keepdims=True)
    dot2 = lax.dot_general(emb2T_ref[...], xall, (((0,), (0,)), ((), ())),
                           preferred_element_type=jnp.float32)
    d = (sx + se) + dot2
    m = jnp.min(d, axis=0, keepdims=True)
    cand = jnp.where(d == m, iota, float(NUM_EMB))
    idx_ref[pl.ds(0, BLK)] = jnp.min(cand, axis=0).astype(jnp.int32)


def _chunk_indices(xt, aux, emb2T, c):
    return pl.pallas_call(
        _argmin_body,
        grid=(1,),
        in_specs=[
            pl.BlockSpec((BCH, DIM, S), lambda i, c=c: (c, 0, 0)),
            pl.BlockSpec((NUM_EMB, 128), lambda i: (0, 0)),
            pl.BlockSpec((DIM, NUM_EMB), lambda i: (0, 0)),
        ],
        out_specs=pl.BlockSpec((IDXPAD,), lambda i: (0,)),
        out_shape=jax.ShapeDtypeStruct((IDXPAD,), jnp.int32),
    )(xt, aux, emb2T)


@functools.cache
def _make_gather():
    info = plsc.get_sparse_core_info()
    nw = info.num_cores * info.num_subcores
    b_per_w = BLK // nw

    @functools.partial(
        pl.kernel,
        out_type=jax.ShapeDtypeStruct((BLK, GDIM), jnp.float32),
        mesh=plsc.VectorSubcoreMesh(core_axis_name="c", subcore_axis_name="s"),
        scratch_types=[
            pltpu.VMEM((b_per_w,), jnp.int32),
            pltpu.VMEM((b_per_w, GDIM), jnp.float32),
            pltpu.SemaphoreType.DMA,
        ],
    )
    def _gather_rows(emb_hbm, idx_hbm, out_hbm, idx_v, rows_v, sem):
        wid = lax.axis_index("s") * info.num_cores + lax.axis_index("c")
        base = wid * b_per_w
        pltpu.sync_copy(idx_hbm.at[pl.ds(base, b_per_w)], idx_v)
        pltpu.async_copy(emb_hbm.at[idx_v], rows_v, sem).wait()
        pltpu.sync_copy(rows_v, out_hbm.at[pl.ds(base, b_per_w)])

    return _gather_rows


def kernel(inputs, emb_weight):
    xt = jnp.swapaxes(inputs, 1, 2)
    embT = jnp.swapaxes(emb_weight, 0, 1)
    aux, emb2T, emb_pad = _prep(embT)
    g = _make_gather()
    idxs, quants = [], []
    for ci in range(NCHUNK):
        idx_c = _chunk_indices(xt, aux, emb2T, ci)
        idxs.append(idx_c[:BLK])
        quants.append(g(emb_pad, idx_c))
    idx = jnp.concatenate(idxs).reshape(B, S)
    q64 = jnp.concatenate(quants)[:, :DIM]
    qT = jnp.transpose(q64.reshape(B, S, DIM), (0, 2, 1))
    quantized = jnp.swapaxes(qT, 1, 2)
    return (quantized, idx)
